# baseline (speedup 1.0000x reference)
import jax
import jax.numpy as jnp
from jax import lax
from jax.experimental import pallas as pl
from jax.experimental.pallas import tpu as pltpu
from jax.experimental.pallas import tpu_sc as plsc

N_USERS = 4000
N_ITEMS = 6000
N_NODES = N_USERS + N_ITEMS
EMB_DIM = 256
HALF = 128
N_LAYERS = 3
N_EDGES = 160000
BATCH = 4096
DECAY = 1e-4

NC = 2
NS = 16
NPAD = 10240
EPAD = 163840
E_PER_TILE = EPAD // NS
CHUNK = 128
N_CHUNKS = E_PER_TILE // CHUNK
ROWS_PER_TILE = NPAD // NS
N_WCHUNKS = ROWS_PER_TILE // CHUNK
B_PER_TILE = BATCH // NS
N_BSUB = B_PER_TILE // CHUNK
PAD_SRC0 = 10000
PAD_SRC_SPREAD = 128
PAD_DST0 = 10128
PAD_DST_SPREAD = 112


def _rsqrt16(x):
    i = lax.bitcast_convert_type(x, jnp.int32)
    i = jnp.int32(0x5F3759DF) - lax.shift_right_arithmetic(i, 1)
    y = lax.bitcast_convert_type(i, jnp.float32)
    for _ in range(3):
        y = y * (1.5 - 0.5 * x * y * y)
    return y


def _gcn_body(ego_r, ed_r, bt_r,
              scores_r, regp_r, s_r, tsum_r, f_r, g_r,
              acc, pair0, pair1, pair2, bidx, buf_a, buf_b, fstg, gstg,
              scp, scn, rp,
              isem0, isem1, isem2, gsem0, gsem1, gsem2,
              ssem0, ssem1, ssem2):
    c = lax.axis_index("c")
    s = lax.axis_index("s")
    cbase = c * NPAD
    nr0 = s * ROWS_PER_TILE
    pr0 = s * (ROWS_PER_TILE // 8)
    z16 = jnp.zeros((16,), jnp.float32)
    one16 = jnp.ones((16,), jnp.float32)
    pairs = (pair0, pair1)
    isems = (isem0, isem1)
    pairs3 = (pair0, pair1, pair2)
    isems3 = (isem0, isem1, isem2)
    gsems3 = (gsem0, gsem1, gsem2)
    ssems3 = (ssem0, ssem1, ssem2)
    bufs = (buf_a, buf_b)

    def m8(x):
        return pl.multiple_of(x, 8)

    def ds8(start, size):
        return pl.ds(m8(start), size)

    zrows = ego_r.at[ds8(cbase + PAD_SRC0, CHUNK)]

    def wait_idx(b):
        pltpu.make_async_copy(ed_r.at[s, 0], pairs[b], isems[b]).wait()

    def _init_row(i, _):
        for q in range(8):
            buf_b[i, pl.ds(16 * q, 16)] = one16
        return 0
    lax.fori_loop(0, CHUNK, _init_row, 0)

    for k in range(N_WCHUNKS):
        pltpu.async_copy(zrows, acc.at[ds8(nr0 + CHUNK * k, CHUNK)], gsem0)
    for k in range(N_WCHUNKS):
        pltpu.make_async_copy(zrows, acc.at[ds8(nr0, CHUNK)], gsem0).wait()
    plsc.subcore_barrier()

    for side in range(2):
        f_out = f_r if side == 0 else g_r

        pltpu.async_copy(ed_r.at[s, 0], pair0, isem0)

        def _dwait_i(p):
            pltpu.make_async_copy(ed_r.at[s, 0], pairs3[p], isems3[p]).wait()

        def _dwait_s(b):
            pltpu.make_async_copy(
                buf_b, acc.at[pairs3[0].at[side]], ssems3[b]).wait()

        def _dstep(j, rr, g_sw):
            p, b = rr % 3, rr % 2
            pn = (rr + 1) % 3
            _dwait_i(p)

            def _sw():
                _dwait_s(b)
            if g_sw is None:
                _sw()
            else:
                pl.when(g_sw)(_sw)
            pltpu.async_copy(
                buf_b, acc.at[pairs3[p].at[side]], ssems3[b], add=True)
            pltpu.async_copy(ed_r.at[s, j + 1], pairs3[pn], isems3[pn])

        def _dpipe(i, _):
            _dstep(6 * i + 0, 0, i > 0)
            _dstep(6 * i + 1, 1, i > 0)
            for rr in range(2, 6):
                _dstep(6 * i + rr, rr, None)
            return 0
        lax.fori_loop(0, (N_CHUNKS - 2) // 6, _dpipe, 0)
        _dwait_i(0)
        _dwait_s(0)
        pltpu.async_copy(buf_b, acc.at[pair0.at[side]], ssem0, add=True)
        pltpu.async_copy(ed_r.at[s, N_CHUNKS - 1], pair1, isem1)
        _dwait_i(1)
        _dwait_s(1)
        pltpu.async_copy(buf_b, acc.at[pair1.at[side]], ssem1, add=True)
        _dwait_s(0)
        _dwait_s(1)
        plsc.subcore_barrier()

        def _ext_chunk(k, _):
            pltpu.sync_copy(acc.at[ds8(nr0 + CHUNK * k, CHUNK)], buf_a)

            def _ext_row(i, __):
                fstg[i // 8, pl.ds((i % 8) * 16, 16)] = _rsqrt16(
                    jnp.maximum(buf_a[i, pl.ds(0, 16)], 1.0))
                return 0
            lax.fori_loop(0, CHUNK, _ext_row, 0)
            pltpu.sync_copy(fstg, f_out.at[ds8(pr0 + (CHUNK // 8) * k, CHUNK // 8)])
            pltpu.async_copy(zrows, acc.at[ds8(nr0 + CHUNK * k, CHUNK)], gsem1)
            return 0
        lax.fori_loop(0, N_WCHUNKS, _ext_chunk, 0)

        def _ext_drain(k, _):
            pltpu.make_async_copy(zrows, acc.at[ds8(nr0, CHUNK)], gsem1).wait()
            return 0
        lax.fori_loop(0, N_WCHUNKS, _ext_drain, 0)
        plsc.subcore_barrier()

    def _b_chunk(k, ra):
        r0 = nr0 + CHUNK * k
        pltpu.async_copy(ego_r.at[ds8(cbase + r0, CHUNK)], buf_a, gsem0)
        pltpu.async_copy(
            f_r.at[ds8(pr0 + (CHUNK // 8) * k, CHUNK // 8)], fstg, isem0)
        pltpu.make_async_copy(ego_r.at[ds8(cbase + r0, CHUNK)], buf_a, gsem0).wait()
        pltpu.make_async_copy(
            f_r.at[ds8(pr0, CHUNK // 8)], fstg, isem0).wait()

        def _b_row(i, rr):
            fsp = fstg[i // 8, pl.ds((i % 8) * 16, 16)]
            for q in range(8):
                sl = pl.ds(16 * q, 16)
                a = buf_a[i, sl]
                rr = rr + a * a
                buf_b[i, sl] = a * fsp
            return rr
        ra = lax.fori_loop(0, CHUNK, _b_row, ra)
        pltpu.async_copy(buf_a, tsum_r.at[ds8(cbase + r0, CHUNK)], gsem0)
        pltpu.async_copy(buf_b, s_r.at[ds8(cbase + r0, CHUNK)], gsem1)
        pltpu.make_async_copy(buf_a, tsum_r.at[ds8(cbase + r0, CHUNK)], gsem0).wait()
        pltpu.make_async_copy(buf_b, s_r.at[ds8(cbase + r0, CHUNK)], gsem1).wait()
        return ra
    racc = lax.fori_loop(0, N_WCHUNKS, _b_chunk, z16)
    rp[pl.ds(0, 16)] = racc
    pltpu.sync_copy(rp, regp_r.at[ds8((c * NS + s) * 16, 16)])
    plsc.subcore_barrier()

    for l in range(N_LAYERS):
        last = (l == N_LAYERS - 1)

        pltpu.async_copy(ed_r.at[s, 0], pair0, isem0)

        def _wait_i(p):
            pltpu.make_async_copy(ed_r.at[s, 0], pairs3[p], isems3[p]).wait()

        def _wait_g(b):
            pltpu.make_async_copy(
                s_r.at[pairs3[0].at[0]], bufs[b], gsems3[b]).wait()

        def _wait_s(b):
            pltpu.make_async_copy(
                bufs[b], acc.at[pairs3[0].at[1]], ssems3[b]).wait()

        def _step(j, rr, g_sw, g_prev):
            p, b = rr % 3, rr % 2
            pn, bo = (rr + 1) % 3, 1 - (rr % 2)
            _wait_i(p)
            pr = pairs3[p]
            for q in range(8):
                sl = pl.ds(16 * q, 16)
                pr[0, sl] = pr[0, sl] + cbase

            def _sw():
                _wait_s(b)
            if g_sw is None:
                _sw()
            else:
                pl.when(g_sw)(_sw)
            pltpu.async_copy(s_r.at[pr.at[0]], bufs[b], gsems3[b])

            def _prev():
                _wait_g(bo)
                pltpu.async_copy(
                    bufs[bo], acc.at[pairs3[(rr + 2) % 3].at[1]], ssems3[bo],
                    add=True)
            if g_prev is None:
                _prev()
            else:
                pl.when(g_prev)(_prev)
            pltpu.async_copy(ed_r.at[s, j + 1], pairs3[pn], isems3[pn])

        def _pipe(i, _):
            _step(6 * i + 0, 0, i > 0, i > 0)
            _step(6 * i + 1, 1, i > 0, None)
            for rr in range(2, 6):
                _step(6 * i + rr, rr, None, None)
            return 0
        lax.fori_loop(0, (N_CHUNKS - 2) // 6, _pipe, 0)
        _wait_i(0)
        for q in range(8):
            sl = pl.ds(16 * q, 16)
            pair0[0, sl] = pair0[0, sl] + cbase
        _wait_s(0)
        pltpu.async_copy(s_r.at[pair0.at[0]], buf_a, gsem0)
        _wait_g(1)
        pltpu.async_copy(buf_b, acc.at[pair2.at[1]], ssem1, add=True)
        pltpu.async_copy(ed_r.at[s, N_CHUNKS - 1], pair1, isem1)
        _wait_i(1)
        for q in range(8):
            sl = pl.ds(16 * q, 16)
            pair1[0, sl] = pair1[0, sl] + cbase
        _wait_s(1)
        pltpu.async_copy(s_r.at[pair1.at[0]], buf_b, gsem1)
        _wait_g(0)
        pltpu.async_copy(buf_a, acc.at[pair0.at[1]], ssem0, add=True)
        _wait_g(1)
        pltpu.async_copy(buf_b, acc.at[pair1.at[1]], ssem1, add=True)
        _wait_s(0)
        _wait_s(1)
        plsc.subcore_barrier()

        def _w_chunk(k, _):
            r0 = nr0 + CHUNK * k
            kp = ds8(pr0 + (CHUNK // 8) * k, CHUNK // 8)
            pltpu.async_copy(acc.at[ds8(r0, CHUNK)], buf_a, gsem0)
            pltpu.async_copy(tsum_r.at[ds8(cbase + r0, CHUNK)], buf_b, gsem1)
            pltpu.async_copy(f_r.at[kp], fstg, isem0)
            pltpu.async_copy(g_r.at[kp], gstg, isem1)
            pltpu.make_async_copy(acc.at[ds8(r0, CHUNK)], buf_a, gsem0).wait()
            pltpu.make_async_copy(
                tsum_r.at[ds8(cbase + r0, CHUNK)], buf_b, gsem1).wait()
            pltpu.make_async_copy(f_r.at[ds8(pr0, CHUNK // 8)], fstg, isem0).wait()
            pltpu.make_async_copy(g_r.at[ds8(pr0, CHUNK // 8)], gstg, isem1).wait()

            def _w_row(i, __):
                sl8 = pl.ds((i % 8) * 16, 16)
                gsp = gstg[i // 8, sl8]
                fsp = fstg[i // 8, sl8]
                for q in range(8):
                    sl = pl.ds(16 * q, 16)
                    t = buf_a[i, sl] * gsp
                    buf_b[i, sl] = buf_b[i, sl] + t
                    if not last:
                        buf_a[i, sl] = t * fsp
                return 0
            lax.fori_loop(0, CHUNK, _w_row, 0)
            pltpu.async_copy(buf_b, tsum_r.at[ds8(cbase + r0, CHUNK)], gsem0)
            if not last:
                pltpu.async_copy(buf_a, s_r.at[ds8(cbase + r0, CHUNK)], gsem1)
                pltpu.async_copy(zrows, acc.at[ds8(r0, CHUNK)], isem0)
            pltpu.make_async_copy(
                buf_b, tsum_r.at[ds8(cbase + r0, CHUNK)], gsem0).wait()
            if not last:
                pltpu.make_async_copy(
                    buf_a, s_r.at[ds8(cbase + r0, CHUNK)], gsem1).wait()
                pltpu.make_async_copy(zrows, acc.at[ds8(r0, CHUNK)], isem0).wait()
            return 0
        lax.fori_loop(0, N_WCHUNKS, _w_chunk, 0)
        plsc.subcore_barrier()

    for sub in range(N_BSUB):
        out0 = c * 2 * BATCH + s * B_PER_TILE + sub * CHUNK

        pltpu.sync_copy(bt_r.at[s, sub], bidx)
        for r in range(3):
            for q in range(8):
                sl = pl.ds(16 * q, 16)
                bidx[r, sl] = bidx[r, sl] + cbase
        pltpu.async_copy(tsum_r.at[bidx.at[0]], buf_a, gsem0)
        pltpu.async_copy(tsum_r.at[bidx.at[1]], buf_b, gsem1)
        pltpu.make_async_copy(tsum_r.at[bidx.at[0]], buf_a, gsem0).wait()
        pltpu.make_async_copy(tsum_r.at[bidx.at[1]], buf_b, gsem1).wait()

        def _prow(i, _):
            dp = z16
            for q in range(8):
                sl = pl.ds(16 * q, 16)
                dp = dp + buf_a[i, sl] * buf_b[i, sl]
            scp[i // 8, pl.ds((i % 8) * 16, 16)] = dp
            return 0
        lax.fori_loop(0, CHUNK, _prow, 0)

        pltpu.sync_copy(tsum_r.at[bidx.at[2]], buf_b)

        def _nrow(i, _):
            dn = z16
            for q in range(8):
                sl = pl.ds(16 * q, 16)
                dn = dn + buf_a[i, sl] * buf_b[i, sl]
            scn[i // 8, pl.ds((i % 8) * 16, 16)] = dn
            return 0
        lax.fori_loop(0, CHUNK, _nrow, 0)

        pltpu.async_copy(scp, scores_r.at[ds8(out0 // 8, CHUNK // 8)], gsem0)
        pltpu.async_copy(
            scn, scores_r.at[ds8((out0 + BATCH) // 8, CHUNK // 8)], gsem1)
        pltpu.make_async_copy(
            scp, scores_r.at[ds8(out0 // 8, CHUNK // 8)], gsem0).wait()
        pltpu.make_async_copy(
            scn, scores_r.at[ds8((out0 + BATCH) // 8, CHUNK // 8)], gsem1).wait()


@jax.jit
def _gcn(ego2, ed, bt):
    mesh = plsc.VectorSubcoreMesh(core_axis_name="c", subcore_axis_name="s")
    f32 = jnp.float32
    return pl.kernel(
        _gcn_body,
        out_type=[
            jax.ShapeDtypeStruct((2 * 2 * BATCH // 8, 128), f32),
            jax.ShapeDtypeStruct((NC * NS * 16,), f32),
            jax.ShapeDtypeStruct((2 * NPAD, HALF), f32),
            jax.ShapeDtypeStruct((2 * NPAD, HALF), f32),
            jax.ShapeDtypeStruct((NPAD // 8, 128), f32),
            jax.ShapeDtypeStruct((NPAD // 8, 128), f32),
        ],
        mesh=mesh,
        scratch_types=[
            pltpu.VMEM_SHARED((NPAD, HALF), f32),
            pltpu.VMEM((2, CHUNK), jnp.int32),
            pltpu.VMEM((2, CHUNK), jnp.int32),
            pltpu.VMEM((2, CHUNK), jnp.int32),
            pltpu.VMEM((3, CHUNK), jnp.int32),
            pltpu.VMEM((CHUNK, HALF), f32),
            pltpu.VMEM((CHUNK, HALF), f32),
            pltpu.VMEM((CHUNK // 8, 128), f32),
            pltpu.VMEM((CHUNK // 8, 128), f32),
            pltpu.VMEM((CHUNK // 8, 128), f32),
            pltpu.VMEM((CHUNK // 8, 128), f32),
            pltpu.VMEM((16,), f32),
            pltpu.SemaphoreType.DMA,
            pltpu.SemaphoreType.DMA,
            pltpu.SemaphoreType.DMA,
            pltpu.SemaphoreType.DMA,
            pltpu.SemaphoreType.DMA,
            pltpu.SemaphoreType.DMA,
            pltpu.SemaphoreType.DMA,
            pltpu.SemaphoreType.DMA,
            pltpu.SemaphoreType.DMA,
        ],
    )(ego2, ed, bt)


def kernel(users, pos_items, neg_items, edge_index, user_embedding,
           item_embedding):
    ego = jnp.zeros((NPAD, EMB_DIM), jnp.float32)
    ego = ego.at[:N_USERS].set(user_embedding)
    ego = ego.at[N_USERS:N_NODES].set(item_embedding)
    ego2 = jnp.concatenate([ego[:, :HALF], ego[:, HALF:]], axis=0)

    pad = jnp.arange(EPAD - N_EDGES, dtype=jnp.int32)
    srcp = jnp.concatenate(
        [edge_index[0], PAD_SRC0 + pad % PAD_SRC_SPREAD]
    ).reshape(NS, N_CHUNKS, CHUNK)
    dstp = jnp.concatenate(
        [edge_index[1], PAD_DST0 + pad % PAD_DST_SPREAD]
    ).reshape(NS, N_CHUNKS, CHUNK)
    ed = jnp.stack([srcp, dstp], axis=2)

    ur = users.reshape(NS, N_BSUB, CHUNK)
    pr = (pos_items + N_USERS).reshape(NS, N_BSUB, CHUNK)
    nr = (neg_items + N_USERS).reshape(NS, N_BSUB, CHUNK)
    bt = jnp.stack([ur, pr, nr], axis=2)

    scores, regp, _, _, _, _ = _gcn(ego2, ed, bt)
    sc = jnp.sum(scores.reshape(2, 2, BATCH // 8, 8, 16), axis=-1).reshape(2, 2, BATCH)
    pos_s = (sc[0, 0] + sc[1, 0]) * (1.0 / 16.0)
    neg_s = (sc[0, 1] + sc[1, 1]) * (1.0 / 16.0)
    mf_loss = jnp.mean(jax.nn.softplus(-(pos_s - neg_s)))
    emb_loss = DECAY * (0.5 * jnp.sum(regp) / BATCH)
    return mf_loss + emb_loss


# speedup vs baseline: 1.0097x; 1.0097x over previous
import jax
import jax.numpy as jnp
from jax import lax
from jax.experimental import pallas as pl
from jax.experimental.pallas import tpu as pltpu
from jax.experimental.pallas import tpu_sc as plsc

N_USERS = 4000
N_ITEMS = 6000
N_NODES = N_USERS + N_ITEMS
EMB_DIM = 256
HALF = 128
N_LAYERS = 3
N_EDGES = 160000
BATCH = 4096
DECAY = 1e-4

NC = 2
NS = 16
NPAD = 10240
EPAD = 163840
E_PER_TILE = EPAD // NS
CHUNK = 128
N_CHUNKS = E_PER_TILE // CHUNK
ROWS_PER_TILE = NPAD // NS
N_WCHUNKS = ROWS_PER_TILE // CHUNK
B_PER_TILE = BATCH // NS
N_BSUB = B_PER_TILE // CHUNK
PAD_SRC0 = 10000
PAD_SRC_SPREAD = 128
PAD_DST0 = 10128
PAD_DST_SPREAD = 112


def _rsqrt16(x):
    i = lax.bitcast_convert_type(x, jnp.int32)
    i = jnp.int32(0x5F3759DF) - lax.shift_right_arithmetic(i, 1)
    y = lax.bitcast_convert_type(i, jnp.float32)
    for _ in range(3):
        y = y * (1.5 - 0.5 * x * y * y)
    return y


def _gcn_body(ego_r, ed_r, bt_r,
              scores_r, regp_r, s_r, tsum_r, f_r, g_r,
              acc, pair0, pair1, pair2, bidx, buf_a, buf_b, fstg, gstg,
              scp, scn, rp,
              isem0, isem1, isem2, gsem0, gsem1, gsem2,
              ssem0, ssem1, ssem2):
    c = lax.axis_index("c")
    s = lax.axis_index("s")
    cbase = c * NPAD
    nr0 = s * ROWS_PER_TILE
    pr0 = s * (ROWS_PER_TILE // 8)
    z16 = jnp.zeros((16,), jnp.float32)
    one16 = jnp.ones((16,), jnp.float32)
    pairs = (pair0, pair1)
    isems = (isem0, isem1)
    pairs3 = (pair0, pair1, pair2)
    isems3 = (isem0, isem1, isem2)
    gsems3 = (gsem0, gsem1, gsem2)
    ssems3 = (ssem0, ssem1, ssem2)
    bufs = (buf_a, buf_b)

    def m8(x):
        return pl.multiple_of(x, 8)

    def ds8(start, size):
        return pl.ds(m8(start), size)

    zrows = ego_r.at[ds8(cbase + PAD_SRC0, CHUNK)]

    def wait_idx(b):
        pltpu.make_async_copy(ed_r.at[s, 0], pairs[b], isems[b]).wait()

    def _init_row(i, _):
        for q in range(8):
            buf_b[i, pl.ds(16 * q, 16)] = one16 if q < 4 else z16
            buf_a[i, pl.ds(16 * q, 16)] = z16 if q < 4 else one16
        return 0
    lax.fori_loop(0, CHUNK, _init_row, 0)

    for k in range(N_WCHUNKS):
        pltpu.async_copy(zrows, acc.at[ds8(nr0 + CHUNK * k, CHUNK)], gsem0)
    for k in range(N_WCHUNKS):
        pltpu.make_async_copy(zrows, acc.at[ds8(nr0, CHUNK)], gsem0).wait()
    plsc.subcore_barrier()

    pltpu.async_copy(ed_r.at[s, 0], pair0, isem0)

    def _dwait_i(p):
        pltpu.make_async_copy(ed_r.at[s, 0], pairs3[p], isems3[p]).wait()

    def _dwait_s(b):
        pltpu.make_async_copy(
            buf_b, acc.at[pairs3[0].at[0]], ssems3[b]).wait()

    def _dwait_d(b):
        pltpu.make_async_copy(
            buf_a, acc.at[pairs3[0].at[1]], gsems3[b]).wait()

    def _dstep(j, rr, g_sw):
        p, b = rr % 3, rr % 2
        pn = (rr + 1) % 3
        _dwait_i(p)

        def _sw():
            _dwait_s(b)
            _dwait_d(b)
        if g_sw is None:
            _sw()
        else:
            pl.when(g_sw)(_sw)
        pltpu.async_copy(
            buf_b, acc.at[pairs3[p].at[0]], ssems3[b], add=True)
        pltpu.async_copy(
            buf_a, acc.at[pairs3[p].at[1]], gsems3[b], add=True)
        pltpu.async_copy(ed_r.at[s, j + 1], pairs3[pn], isems3[pn])

    def _dpipe(i, _):
        _dstep(6 * i + 0, 0, i > 0)
        _dstep(6 * i + 1, 1, i > 0)
        for rr in range(2, 6):
            _dstep(6 * i + rr, rr, None)
        return 0
    lax.fori_loop(0, (N_CHUNKS - 2) // 6, _dpipe, 0)
    _dwait_i(0)
    _dwait_s(0)
    _dwait_d(0)
    pltpu.async_copy(buf_b, acc.at[pair0.at[0]], ssem0, add=True)
    pltpu.async_copy(buf_a, acc.at[pair0.at[1]], gsem0, add=True)
    pltpu.async_copy(ed_r.at[s, N_CHUNKS - 1], pair1, isem1)
    _dwait_i(1)
    _dwait_s(1)
    _dwait_d(1)
    pltpu.async_copy(buf_b, acc.at[pair1.at[0]], ssem1, add=True)
    pltpu.async_copy(buf_a, acc.at[pair1.at[1]], gsem1, add=True)
    _dwait_s(0)
    _dwait_d(0)
    _dwait_s(1)
    _dwait_d(1)
    plsc.subcore_barrier()

    def _ext_chunk(k, _):
        pltpu.sync_copy(acc.at[ds8(nr0 + CHUNK * k, CHUNK)], buf_a)

        def _ext_row(i, __):
            sl8 = pl.ds((i % 8) * 16, 16)
            fstg[i // 8, sl8] = _rsqrt16(
                jnp.maximum(buf_a[i, pl.ds(0, 16)], 1.0))
            gstg[i // 8, sl8] = _rsqrt16(
                jnp.maximum(buf_a[i, pl.ds(64, 16)], 1.0))
            return 0
        lax.fori_loop(0, CHUNK, _ext_row, 0)
        kp = ds8(pr0 + (CHUNK // 8) * k, CHUNK // 8)
        pltpu.sync_copy(fstg, f_r.at[kp])
        pltpu.sync_copy(gstg, g_r.at[kp])
        pltpu.async_copy(zrows, acc.at[ds8(nr0 + CHUNK * k, CHUNK)], gsem1)
        return 0
    lax.fori_loop(0, N_WCHUNKS, _ext_chunk, 0)

    def _ext_drain(k, _):
        pltpu.make_async_copy(zrows, acc.at[ds8(nr0, CHUNK)], gsem1).wait()
        return 0
    lax.fori_loop(0, N_WCHUNKS, _ext_drain, 0)
    plsc.subcore_barrier()

    def _b_chunk(k, ra):
        r0 = nr0 + CHUNK * k
        pltpu.async_copy(ego_r.at[ds8(cbase + r0, CHUNK)], buf_a, gsem0)
        pltpu.async_copy(
            f_r.at[ds8(pr0 + (CHUNK // 8) * k, CHUNK // 8)], fstg, isem0)
        pltpu.make_async_copy(ego_r.at[ds8(cbase + r0, CHUNK)], buf_a, gsem0).wait()
        pltpu.make_async_copy(
            f_r.at[ds8(pr0, CHUNK // 8)], fstg, isem0).wait()

        def _b_row(i, rr):
            fsp = fstg[i // 8, pl.ds((i % 8) * 16, 16)]
            for q in range(8):
                sl = pl.ds(16 * q, 16)
                a = buf_a[i, sl]
                rr = rr + a * a
                buf_b[i, sl] = a * fsp
            return rr
        ra = lax.fori_loop(0, CHUNK, _b_row, ra)
        pltpu.async_copy(buf_a, tsum_r.at[ds8(cbase + r0, CHUNK)], gsem0)
        pltpu.async_copy(buf_b, s_r.at[ds8(cbase + r0, CHUNK)], gsem1)
        pltpu.make_async_copy(buf_a, tsum_r.at[ds8(cbase + r0, CHUNK)], gsem0).wait()
        pltpu.make_async_copy(buf_b, s_r.at[ds8(cbase + r0, CHUNK)], gsem1).wait()
        return ra
    racc = lax.fori_loop(0, N_WCHUNKS, _b_chunk, z16)
    rp[pl.ds(0, 16)] = racc
    pltpu.sync_copy(rp, regp_r.at[ds8((c * NS + s) * 16, 16)])
    plsc.subcore_barrier()

    for l in range(N_LAYERS):
        last = (l == N_LAYERS - 1)

        pltpu.async_copy(ed_r.at[s, 0], pair0, isem0)

        def _wait_i(p):
            pltpu.make_async_copy(ed_r.at[s, 0], pairs3[p], isems3[p]).wait()

        def _wait_g(b):
            pltpu.make_async_copy(
                s_r.at[pairs3[0].at[0]], bufs[b], gsems3[b]).wait()

        def _wait_s(b):
            pltpu.make_async_copy(
                bufs[b], acc.at[pairs3[0].at[1]], ssems3[b]).wait()

        def _step(j, rr, g_sw, g_prev):
            p, b = rr % 3, rr % 2
            pn, bo = (rr + 1) % 3, 1 - (rr % 2)
            _wait_i(p)
            pr = pairs3[p]
            for q in range(8):
                sl = pl.ds(16 * q, 16)
                pr[0, sl] = pr[0, sl] + cbase

            def _sw():
                _wait_s(b)
            if g_sw is None:
                _sw()
            else:
                pl.when(g_sw)(_sw)
            pltpu.async_copy(s_r.at[pr.at[0]], bufs[b], gsems3[b])

            def _prev():
                _wait_g(bo)
                pltpu.async_copy(
                    bufs[bo], acc.at[pairs3[(rr + 2) % 3].at[1]], ssems3[bo],
                    add=True)
            if g_prev is None:
                _prev()
            else:
                pl.when(g_prev)(_prev)
            pltpu.async_copy(ed_r.at[s, j + 1], pairs3[pn], isems3[pn])

        def _pipe(i, _):
            _step(6 * i + 0, 0, i > 0, i > 0)
            _step(6 * i + 1, 1, i > 0, None)
            for rr in range(2, 6):
                _step(6 * i + rr, rr, None, None)
            return 0
        lax.fori_loop(0, (N_CHUNKS - 2) // 6, _pipe, 0)
        _wait_i(0)
        for q in range(8):
            sl = pl.ds(16 * q, 16)
            pair0[0, sl] = pair0[0, sl] + cbase
        _wait_s(0)
        pltpu.async_copy(s_r.at[pair0.at[0]], buf_a, gsem0)
        _wait_g(1)
        pltpu.async_copy(buf_b, acc.at[pair2.at[1]], ssem1, add=True)
        pltpu.async_copy(ed_r.at[s, N_CHUNKS - 1], pair1, isem1)
        _wait_i(1)
        for q in range(8):
            sl = pl.ds(16 * q, 16)
            pair1[0, sl] = pair1[0, sl] + cbase
        _wait_s(1)
        pltpu.async_copy(s_r.at[pair1.at[0]], buf_b, gsem1)
        _wait_g(0)
        pltpu.async_copy(buf_a, acc.at[pair0.at[1]], ssem0, add=True)
        _wait_g(1)
        pltpu.async_copy(buf_b, acc.at[pair1.at[1]], ssem1, add=True)
        _wait_s(0)
        _wait_s(1)
        plsc.subcore_barrier()

        def _w_chunk(k, _):
            r0 = nr0 + CHUNK * k
            kp = ds8(pr0 + (CHUNK // 8) * k, CHUNK // 8)
            pltpu.async_copy(acc.at[ds8(r0, CHUNK)], buf_a, gsem0)
            pltpu.async_copy(tsum_r.at[ds8(cbase + r0, CHUNK)], buf_b, gsem1)
            pltpu.async_copy(f_r.at[kp], fstg, isem0)
            pltpu.async_copy(g_r.at[kp], gstg, isem1)
            pltpu.make_async_copy(acc.at[ds8(r0, CHUNK)], buf_a, gsem0).wait()
            pltpu.make_async_copy(
                tsum_r.at[ds8(cbase + r0, CHUNK)], buf_b, gsem1).wait()
            pltpu.make_async_copy(f_r.at[ds8(pr0, CHUNK // 8)], fstg, isem0).wait()
            pltpu.make_async_copy(g_r.at[ds8(pr0, CHUNK // 8)], gstg, isem1).wait()

            def _w_row(i, __):
                sl8 = pl.ds((i % 8) * 16, 16)
                gsp = gstg[i // 8, sl8]
                fsp = fstg[i // 8, sl8]
                for q in range(8):
                    sl = pl.ds(16 * q, 16)
                    t = buf_a[i, sl] * gsp
                    buf_b[i, sl] = buf_b[i, sl] + t
                    if not last:
                        buf_a[i, sl] = t * fsp
                return 0
            lax.fori_loop(0, CHUNK, _w_row, 0)
            pltpu.async_copy(buf_b, tsum_r.at[ds8(cbase + r0, CHUNK)], gsem0)
            if not last:
                pltpu.async_copy(buf_a, s_r.at[ds8(cbase + r0, CHUNK)], gsem1)
                pltpu.async_copy(zrows, acc.at[ds8(r0, CHUNK)], isem0)
            pltpu.make_async_copy(
                buf_b, tsum_r.at[ds8(cbase + r0, CHUNK)], gsem0).wait()
            if not last:
                pltpu.make_async_copy(
                    buf_a, s_r.at[ds8(cbase + r0, CHUNK)], gsem1).wait()
                pltpu.make_async_copy(zrows, acc.at[ds8(r0, CHUNK)], isem0).wait()
            return 0
        lax.fori_loop(0, N_WCHUNKS, _w_chunk, 0)
        plsc.subcore_barrier()

    for sub in range(N_BSUB):
        out0 = c * 2 * BATCH + s * B_PER_TILE + sub * CHUNK

        pltpu.sync_copy(bt_r.at[s, sub], bidx)
        for r in range(3):
            for q in range(8):
                sl = pl.ds(16 * q, 16)
                bidx[r, sl] = bidx[r, sl] + cbase
        pltpu.async_copy(tsum_r.at[bidx.at[0]], buf_a, gsem0)
        pltpu.async_copy(tsum_r.at[bidx.at[1]], buf_b, gsem1)
        pltpu.make_async_copy(tsum_r.at[bidx.at[0]], buf_a, gsem0).wait()
        pltpu.make_async_copy(tsum_r.at[bidx.at[1]], buf_b, gsem1).wait()

        def _prow(i, _):
            dp = z16
            for q in range(8):
                sl = pl.ds(16 * q, 16)
                dp = dp + buf_a[i, sl] * buf_b[i, sl]
            scp[i // 8, pl.ds((i % 8) * 16, 16)] = dp
            return 0
        lax.fori_loop(0, CHUNK, _prow, 0)

        pltpu.sync_copy(tsum_r.at[bidx.at[2]], buf_b)

        def _nrow(i, _):
            dn = z16
            for q in range(8):
                sl = pl.ds(16 * q, 16)
                dn = dn + buf_a[i, sl] * buf_b[i, sl]
            scn[i // 8, pl.ds((i % 8) * 16, 16)] = dn
            return 0
        lax.fori_loop(0, CHUNK, _nrow, 0)

        pltpu.async_copy(scp, scores_r.at[ds8(out0 // 8, CHUNK // 8)], gsem0)
        pltpu.async_copy(
            scn, scores_r.at[ds8((out0 + BATCH) // 8, CHUNK // 8)], gsem1)
        pltpu.make_async_copy(
            scp, scores_r.at[ds8(out0 // 8, CHUNK // 8)], gsem0).wait()
        pltpu.make_async_copy(
            scn, scores_r.at[ds8((out0 + BATCH) // 8, CHUNK // 8)], gsem1).wait()


@jax.jit
def _gcn(ego2, ed, bt):
    mesh = plsc.VectorSubcoreMesh(core_axis_name="c", subcore_axis_name="s")
    f32 = jnp.float32
    return pl.kernel(
        _gcn_body,
        out_type=[
            jax.ShapeDtypeStruct((2 * 2 * BATCH // 8, 128), f32),
            jax.ShapeDtypeStruct((NC * NS * 16,), f32),
            jax.ShapeDtypeStruct((2 * NPAD, HALF), f32),
            jax.ShapeDtypeStruct((2 * NPAD, HALF), f32),
            jax.ShapeDtypeStruct((NPAD // 8, 128), f32),
            jax.ShapeDtypeStruct((NPAD // 8, 128), f32),
        ],
        mesh=mesh,
        scratch_types=[
            pltpu.VMEM_SHARED((NPAD, HALF), f32),
            pltpu.VMEM((2, CHUNK), jnp.int32),
            pltpu.VMEM((2, CHUNK), jnp.int32),
            pltpu.VMEM((2, CHUNK), jnp.int32),
            pltpu.VMEM((3, CHUNK), jnp.int32),
            pltpu.VMEM((CHUNK, HALF), f32),
            pltpu.VMEM((CHUNK, HALF), f32),
            pltpu.VMEM((CHUNK // 8, 128), f32),
            pltpu.VMEM((CHUNK // 8, 128), f32),
            pltpu.VMEM((CHUNK // 8, 128), f32),
            pltpu.VMEM((CHUNK // 8, 128), f32),
            pltpu.VMEM((16,), f32),
            pltpu.SemaphoreType.DMA,
            pltpu.SemaphoreType.DMA,
            pltpu.SemaphoreType.DMA,
            pltpu.SemaphoreType.DMA,
            pltpu.SemaphoreType.DMA,
            pltpu.SemaphoreType.DMA,
            pltpu.SemaphoreType.DMA,
            pltpu.SemaphoreType.DMA,
            pltpu.SemaphoreType.DMA,
        ],
    )(ego2, ed, bt)


def kernel(users, pos_items, neg_items, edge_index, user_embedding,
           item_embedding):
    ego = jnp.zeros((NPAD, EMB_DIM), jnp.float32)
    ego = ego.at[:N_USERS].set(user_embedding)
    ego = ego.at[N_USERS:N_NODES].set(item_embedding)
    ego2 = jnp.concatenate([ego[:, :HALF], ego[:, HALF:]], axis=0)

    pad = jnp.arange(EPAD - N_EDGES, dtype=jnp.int32)
    srcp = jnp.concatenate(
        [edge_index[0], PAD_SRC0 + pad % PAD_SRC_SPREAD]
    ).reshape(NS, N_CHUNKS, CHUNK)
    dstp = jnp.concatenate(
        [edge_index[1], PAD_DST0 + pad % PAD_DST_SPREAD]
    ).reshape(NS, N_CHUNKS, CHUNK)
    ed = jnp.stack([srcp, dstp], axis=2)

    ur = users.reshape(NS, N_BSUB, CHUNK)
    pr = (pos_items + N_USERS).reshape(NS, N_BSUB, CHUNK)
    nr = (neg_items + N_USERS).reshape(NS, N_BSUB, CHUNK)
    bt = jnp.stack([ur, pr, nr], axis=2)

    scores, regp, _, _, _, _ = _gcn(ego2, ed, bt)
    sc = jnp.sum(scores.reshape(2, 2, BATCH // 8, 8, 16), axis=-1).reshape(2, 2, BATCH)
    pos_s = (sc[0, 0] + sc[1, 0]) * (1.0 / 16.0)
    neg_s = (sc[0, 1] + sc[1, 1]) * (1.0 / 16.0)
    mf_loss = jnp.mean(jax.nn.softplus(-(pos_s - neg_s)))
    emb_loss = DECAY * (0.5 * jnp.sum(regp) / BATCH)
    return mf_loss + emb_loss


# speedup vs baseline: 1.0423x; 1.0323x over previous
import jax
import jax.numpy as jnp
from jax import lax
from jax.experimental import pallas as pl
from jax.experimental.pallas import tpu as pltpu
from jax.experimental.pallas import tpu_sc as plsc

N_USERS = 4000
N_ITEMS = 6000
N_NODES = N_USERS + N_ITEMS
EMB_DIM = 256
HALF = 128
N_LAYERS = 3
N_EDGES = 160000
BATCH = 4096
DECAY = 1e-4

NC = 2
NS = 16
NPAD = 10240
EPAD = 163840
E_PER_TILE = EPAD // NS
CHUNK = 128
N_CHUNKS = E_PER_TILE // CHUNK
ROWS_PER_TILE = NPAD // NS
N_WCHUNKS = ROWS_PER_TILE // CHUNK
B_PER_TILE = BATCH // NS
N_BSUB = B_PER_TILE // CHUNK
PAD_SRC0 = 10000
PAD_SRC_SPREAD = 128
PAD_DST0 = 10128
PAD_DST_SPREAD = 112


def _rsqrt16(x):
    i = lax.bitcast_convert_type(x, jnp.int32)
    i = jnp.int32(0x5F3759DF) - lax.shift_right_arithmetic(i, 1)
    y = lax.bitcast_convert_type(i, jnp.float32)
    for _ in range(3):
        y = y * (1.5 - 0.5 * x * y * y)
    return y


def _gcn_body(ego_r, ed_r, bt_r,
              scores_r, regp_r, s_r, tsum_r, f_r, g_r,
              acc, pair0, pair1, pair2, bidx, buf_a, buf_b, fstg, gstg,
              scp, scn, rp,
              isem0, isem1, isem2, gsem0, gsem1, gsem2,
              ssem0, ssem1, ssem2):
    c = lax.axis_index("c")
    s = lax.axis_index("s")
    cbase = c * NPAD
    nr0 = s * ROWS_PER_TILE
    pr0 = s * (ROWS_PER_TILE // 8)
    z16 = jnp.zeros((16,), jnp.float32)
    one16 = jnp.ones((16,), jnp.float32)
    pairs = (pair0, pair1)
    isems = (isem0, isem1)
    pairs3 = (pair0, pair1, pair2)
    isems3 = (isem0, isem1, isem2)
    gsems3 = (gsem0, gsem1, gsem2)
    ssems3 = (ssem0, ssem1, ssem2)
    bufs = (buf_a, buf_b)

    def m8(x):
        return pl.multiple_of(x, 8)

    def ds8(start, size):
        return pl.ds(m8(start), size)

    zrows = ego_r.at[ds8(cbase + PAD_SRC0, CHUNK)]

    def wait_idx(b):
        pltpu.make_async_copy(ed_r.at[s, 0], pairs[b], isems[b]).wait()

    def _init_row(i, _):
        for q in range(8):
            buf_b[i, pl.ds(16 * q, 16)] = one16 if q < 4 else z16
            buf_a[i, pl.ds(16 * q, 16)] = z16 if q < 4 else one16
        return 0
    lax.fori_loop(0, CHUNK, _init_row, 0)

    for k in range(N_WCHUNKS):
        pltpu.async_copy(zrows, acc.at[ds8(nr0 + CHUNK * k, CHUNK)], gsem0)
    for k in range(N_WCHUNKS):
        pltpu.make_async_copy(zrows, acc.at[ds8(nr0, CHUNK)], gsem0).wait()
    plsc.subcore_barrier()

    pltpu.async_copy(ed_r.at[s, 0], pair0, isem0)

    def _dwait_i(p):
        pltpu.make_async_copy(ed_r.at[s, 0], pairs3[p], isems3[p]).wait()

    def _dwait_s(b):
        pltpu.make_async_copy(
            buf_b, acc.at[pairs3[0].at[0]], ssems3[b]).wait()

    def _dwait_d(b):
        pltpu.make_async_copy(
            buf_a, acc.at[pairs3[0].at[1]], gsems3[b]).wait()

    def _dstep(j, rr, g_sw):
        p, b = rr % 3, rr % 2
        pn = (rr + 1) % 3
        _dwait_i(p)

        def _sw():
            _dwait_s(b)
            _dwait_d(b)
        if g_sw is None:
            _sw()
        else:
            pl.when(g_sw)(_sw)
        pltpu.async_copy(
            buf_b, acc.at[pairs3[p].at[0]], ssems3[b], add=True)
        pltpu.async_copy(
            buf_a, acc.at[pairs3[p].at[1]], gsems3[b], add=True)
        pltpu.async_copy(ed_r.at[s, j + 1], pairs3[pn], isems3[pn])

    def _dpipe(i, _):
        _dstep(6 * i + 0, 0, i > 0)
        _dstep(6 * i + 1, 1, i > 0)
        for rr in range(2, 6):
            _dstep(6 * i + rr, rr, None)
        return 0
    lax.fori_loop(0, (N_CHUNKS - 2) // 6, _dpipe, 0)
    _dwait_i(0)
    _dwait_s(0)
    _dwait_d(0)
    pltpu.async_copy(buf_b, acc.at[pair0.at[0]], ssem0, add=True)
    pltpu.async_copy(buf_a, acc.at[pair0.at[1]], gsem0, add=True)
    pltpu.async_copy(ed_r.at[s, N_CHUNKS - 1], pair1, isem1)
    _dwait_i(1)
    _dwait_s(1)
    _dwait_d(1)
    pltpu.async_copy(buf_b, acc.at[pair1.at[0]], ssem1, add=True)
    pltpu.async_copy(buf_a, acc.at[pair1.at[1]], gsem1, add=True)
    _dwait_s(0)
    _dwait_d(0)
    _dwait_s(1)
    _dwait_d(1)
    plsc.subcore_barrier()

    def _eb_chunk(k, ra):
        r0 = nr0 + CHUNK * k
        kp = ds8(pr0 + (CHUNK // 8) * k, CHUNK // 8)
        pltpu.async_copy(acc.at[ds8(r0, CHUNK)], buf_a, gsem0)
        pltpu.async_copy(ego_r.at[ds8(cbase + r0, CHUNK)], buf_b, gsem1)
        pltpu.make_async_copy(acc.at[ds8(r0, CHUNK)], buf_a, gsem0).wait()
        pltpu.make_async_copy(
            ego_r.at[ds8(cbase + r0, CHUNK)], buf_b, gsem1).wait()

        def _eb_row(i, rr):
            sl8 = pl.ds((i % 8) * 16, 16)
            fsp = _rsqrt16(jnp.maximum(buf_a[i, pl.ds(0, 16)], 1.0))
            gsp = _rsqrt16(jnp.maximum(buf_a[i, pl.ds(64, 16)], 1.0))
            fstg[i // 8, sl8] = fsp
            gstg[i // 8, sl8] = gsp
            for q in range(8):
                sl = pl.ds(16 * q, 16)
                a = buf_b[i, sl]
                rr = rr + a * a
                buf_a[i, sl] = a * fsp
            return rr
        ra = lax.fori_loop(0, CHUNK, _eb_row, ra)
        pltpu.async_copy(fstg, f_r.at[kp], isem0)
        pltpu.async_copy(gstg, g_r.at[kp], isem1)
        pltpu.async_copy(buf_b, tsum_r.at[ds8(cbase + r0, CHUNK)], gsem0)
        pltpu.async_copy(buf_a, s_r.at[ds8(cbase + r0, CHUNK)], gsem1)
        pltpu.async_copy(zrows, acc.at[ds8(r0, CHUNK)], isem2)
        pltpu.make_async_copy(fstg, f_r.at[kp], isem0).wait()
        pltpu.make_async_copy(gstg, g_r.at[kp], isem1).wait()
        pltpu.make_async_copy(
            buf_b, tsum_r.at[ds8(cbase + r0, CHUNK)], gsem0).wait()
        pltpu.make_async_copy(
            buf_a, s_r.at[ds8(cbase + r0, CHUNK)], gsem1).wait()
        pltpu.make_async_copy(zrows, acc.at[ds8(r0, CHUNK)], isem2).wait()
        return ra
    racc = lax.fori_loop(0, N_WCHUNKS, _eb_chunk, z16)
    rp[pl.ds(0, 16)] = racc
    pltpu.sync_copy(rp, regp_r.at[ds8((c * NS + s) * 16, 16)])
    plsc.subcore_barrier()

    for l in range(N_LAYERS):
        last = (l == N_LAYERS - 1)

        pltpu.async_copy(ed_r.at[s, 0], pair0, isem0)

        def _wait_i(p):
            pltpu.make_async_copy(ed_r.at[s, 0], pairs3[p], isems3[p]).wait()

        def _wait_g(b):
            pltpu.make_async_copy(
                s_r.at[pairs3[0].at[0]], bufs[b], gsems3[b]).wait()

        def _wait_s(b):
            pltpu.make_async_copy(
                bufs[b], acc.at[pairs3[0].at[1]], ssems3[b]).wait()

        def _step(j, rr, g_sw, g_prev):
            p, b = rr % 3, rr % 2
            pn, bo = (rr + 1) % 3, 1 - (rr % 2)
            _wait_i(p)
            pr = pairs3[p]
            for q in range(8):
                sl = pl.ds(16 * q, 16)
                pr[0, sl] = pr[0, sl] + cbase

            def _sw():
                _wait_s(b)
            if g_sw is None:
                _sw()
            else:
                pl.when(g_sw)(_sw)
            pltpu.async_copy(s_r.at[pr.at[0]], bufs[b], gsems3[b])

            def _prev():
                _wait_g(bo)
                pltpu.async_copy(
                    bufs[bo], acc.at[pairs3[(rr + 2) % 3].at[1]], ssems3[bo],
                    add=True)
            if g_prev is None:
                _prev()
            else:
                pl.when(g_prev)(_prev)
            pltpu.async_copy(ed_r.at[s, j + 1], pairs3[pn], isems3[pn])

        def _pipe(i, _):
            _step(6 * i + 0, 0, i > 0, i > 0)
            _step(6 * i + 1, 1, i > 0, None)
            for rr in range(2, 6):
                _step(6 * i + rr, rr, None, None)
            return 0
        lax.fori_loop(0, (N_CHUNKS - 2) // 6, _pipe, 0)
        _wait_i(0)
        for q in range(8):
            sl = pl.ds(16 * q, 16)
            pair0[0, sl] = pair0[0, sl] + cbase
        _wait_s(0)
        pltpu.async_copy(s_r.at[pair0.at[0]], buf_a, gsem0)
        _wait_g(1)
        pltpu.async_copy(buf_b, acc.at[pair2.at[1]], ssem1, add=True)
        pltpu.async_copy(ed_r.at[s, N_CHUNKS - 1], pair1, isem1)
        _wait_i(1)
        for q in range(8):
            sl = pl.ds(16 * q, 16)
            pair1[0, sl] = pair1[0, sl] + cbase
        _wait_s(1)
        pltpu.async_copy(s_r.at[pair1.at[0]], buf_b, gsem1)
        _wait_g(0)
        pltpu.async_copy(buf_a, acc.at[pair0.at[1]], ssem0, add=True)
        _wait_g(1)
        pltpu.async_copy(buf_b, acc.at[pair1.at[1]], ssem1, add=True)
        _wait_s(0)
        _wait_s(1)
        plsc.subcore_barrier()

        def _w_chunk(k, _):
            r0 = nr0 + CHUNK * k
            kp = ds8(pr0 + (CHUNK // 8) * k, CHUNK // 8)
            pltpu.async_copy(acc.at[ds8(r0, CHUNK)], buf_a, gsem0)
            pltpu.async_copy(tsum_r.at[ds8(cbase + r0, CHUNK)], buf_b, gsem1)
            pltpu.async_copy(f_r.at[kp], fstg, isem0)
            pltpu.async_copy(g_r.at[kp], gstg, isem1)
            pltpu.make_async_copy(acc.at[ds8(r0, CHUNK)], buf_a, gsem0).wait()
            pltpu.make_async_copy(
                tsum_r.at[ds8(cbase + r0, CHUNK)], buf_b, gsem1).wait()
            pltpu.make_async_copy(f_r.at[ds8(pr0, CHUNK // 8)], fstg, isem0).wait()
            pltpu.make_async_copy(g_r.at[ds8(pr0, CHUNK // 8)], gstg, isem1).wait()

            def _w_row(i, __):
                sl8 = pl.ds((i % 8) * 16, 16)
                gsp = gstg[i // 8, sl8]
                fsp = fstg[i // 8, sl8]
                for q in range(8):
                    sl = pl.ds(16 * q, 16)
                    t = buf_a[i, sl] * gsp
                    buf_b[i, sl] = buf_b[i, sl] + t
                    if not last:
                        buf_a[i, sl] = t * fsp
                return 0
            lax.fori_loop(0, CHUNK, _w_row, 0)
            pltpu.async_copy(buf_b, tsum_r.at[ds8(cbase + r0, CHUNK)], gsem0)
            if not last:
                pltpu.async_copy(buf_a, s_r.at[ds8(cbase + r0, CHUNK)], gsem1)
                pltpu.async_copy(zrows, acc.at[ds8(r0, CHUNK)], isem0)
            pltpu.make_async_copy(
                buf_b, tsum_r.at[ds8(cbase + r0, CHUNK)], gsem0).wait()
            if not last:
                pltpu.make_async_copy(
                    buf_a, s_r.at[ds8(cbase + r0, CHUNK)], gsem1).wait()
                pltpu.make_async_copy(zrows, acc.at[ds8(r0, CHUNK)], isem0).wait()
            return 0
        lax.fori_loop(0, N_WCHUNKS, _w_chunk, 0)
        plsc.subcore_barrier()

    for sub in range(N_BSUB):
        out0 = c * 2 * BATCH + s * B_PER_TILE + sub * CHUNK

        pltpu.sync_copy(bt_r.at[s, sub], bidx)
        for r in range(3):
            for q in range(8):
                sl = pl.ds(16 * q, 16)
                bidx[r, sl] = bidx[r, sl] + cbase
        pltpu.async_copy(tsum_r.at[bidx.at[0]], buf_a, gsem0)
        pltpu.async_copy(tsum_r.at[bidx.at[1]], buf_b, gsem1)
        pltpu.make_async_copy(tsum_r.at[bidx.at[0]], buf_a, gsem0).wait()
        pltpu.make_async_copy(tsum_r.at[bidx.at[1]], buf_b, gsem1).wait()

        def _prow(i, _):
            dp = z16
            for q in range(8):
                sl = pl.ds(16 * q, 16)
                dp = dp + buf_a[i, sl] * buf_b[i, sl]
            scp[i // 8, pl.ds((i % 8) * 16, 16)] = dp
            return 0
        lax.fori_loop(0, CHUNK, _prow, 0)

        pltpu.sync_copy(tsum_r.at[bidx.at[2]], buf_b)

        def _nrow(i, _):
            dn = z16
            for q in range(8):
                sl = pl.ds(16 * q, 16)
                dn = dn + buf_a[i, sl] * buf_b[i, sl]
            scn[i // 8, pl.ds((i % 8) * 16, 16)] = dn
            return 0
        lax.fori_loop(0, CHUNK, _nrow, 0)

        pltpu.async_copy(scp, scores_r.at[ds8(out0 // 8, CHUNK // 8)], gsem0)
        pltpu.async_copy(
            scn, scores_r.at[ds8((out0 + BATCH) // 8, CHUNK // 8)], gsem1)
        pltpu.make_async_copy(
            scp, scores_r.at[ds8(out0 // 8, CHUNK // 8)], gsem0).wait()
        pltpu.make_async_copy(
            scn, scores_r.at[ds8((out0 + BATCH) // 8, CHUNK // 8)], gsem1).wait()


@jax.jit
def _gcn(ego2, ed, bt):
    mesh = plsc.VectorSubcoreMesh(core_axis_name="c", subcore_axis_name="s")
    f32 = jnp.float32
    return pl.kernel(
        _gcn_body,
        out_type=[
            jax.ShapeDtypeStruct((2 * 2 * BATCH // 8, 128), f32),
            jax.ShapeDtypeStruct((NC * NS * 16,), f32),
            jax.ShapeDtypeStruct((2 * NPAD, HALF), f32),
            jax.ShapeDtypeStruct((2 * NPAD, HALF), f32),
            jax.ShapeDtypeStruct((NPAD // 8, 128), f32),
            jax.ShapeDtypeStruct((NPAD // 8, 128), f32),
        ],
        mesh=mesh,
        scratch_types=[
            pltpu.VMEM_SHARED((NPAD, HALF), f32),
            pltpu.VMEM((2, CHUNK), jnp.int32),
            pltpu.VMEM((2, CHUNK), jnp.int32),
            pltpu.VMEM((2, CHUNK), jnp.int32),
            pltpu.VMEM((3, CHUNK), jnp.int32),
            pltpu.VMEM((CHUNK, HALF), f32),
            pltpu.VMEM((CHUNK, HALF), f32),
            pltpu.VMEM((CHUNK // 8, 128), f32),
            pltpu.VMEM((CHUNK // 8, 128), f32),
            pltpu.VMEM((CHUNK // 8, 128), f32),
            pltpu.VMEM((CHUNK // 8, 128), f32),
            pltpu.VMEM((16,), f32),
            pltpu.SemaphoreType.DMA,
            pltpu.SemaphoreType.DMA,
            pltpu.SemaphoreType.DMA,
            pltpu.SemaphoreType.DMA,
            pltpu.SemaphoreType.DMA,
            pltpu.SemaphoreType.DMA,
            pltpu.SemaphoreType.DMA,
            pltpu.SemaphoreType.DMA,
            pltpu.SemaphoreType.DMA,
        ],
    )(ego2, ed, bt)


def kernel(users, pos_items, neg_items, edge_index, user_embedding,
           item_embedding):
    ego = jnp.zeros((NPAD, EMB_DIM), jnp.float32)
    ego = ego.at[:N_USERS].set(user_embedding)
    ego = ego.at[N_USERS:N_NODES].set(item_embedding)
    ego2 = jnp.concatenate([ego[:, :HALF], ego[:, HALF:]], axis=0)

    pad = jnp.arange(EPAD - N_EDGES, dtype=jnp.int32)
    srcp = jnp.concatenate(
        [edge_index[0], PAD_SRC0 + pad % PAD_SRC_SPREAD]
    ).reshape(NS, N_CHUNKS, CHUNK)
    dstp = jnp.concatenate(
        [edge_index[1], PAD_DST0 + pad % PAD_DST_SPREAD]
    ).reshape(NS, N_CHUNKS, CHUNK)
    ed = jnp.stack([srcp, dstp], axis=2)

    ur = users.reshape(NS, N_BSUB, CHUNK)
    pr = (pos_items + N_USERS).reshape(NS, N_BSUB, CHUNK)
    nr = (neg_items + N_USERS).reshape(NS, N_BSUB, CHUNK)
    bt = jnp.stack([ur, pr, nr], axis=2)

    scores, regp, _, _, _, _ = _gcn(ego2, ed, bt)
    sc = jnp.sum(scores.reshape(2, 2, BATCH // 8, 8, 16), axis=-1).reshape(2, 2, BATCH)
    pos_s = (sc[0, 0] + sc[1, 0]) * (1.0 / 16.0)
    neg_s = (sc[0, 1] + sc[1, 1]) * (1.0 / 16.0)
    mf_loss = jnp.mean(jax.nn.softplus(-(pos_s - neg_s)))
    emb_loss = DECAY * (0.5 * jnp.sum(regp) / BATCH)
    return mf_loss + emb_loss
